# baseline (speedup 1.0000x reference)
import jax
import jax.numpy as jnp
from jax import lax
from jax.experimental import pallas as pl
from jax.experimental.pallas import tpu as pltpu

DIM = 512
MEM = 100000
ROWS_PER_BLOCK = 2000
NB = MEM // ROWS_PER_BLOCK


def _body(pattern_ref, sp_ref, mean_ref, var_ref, w1_ref, b1_ref, w2_ref,
          b2_ref, out_ref, pmean_ref, part_ref, min_ref):
    i = pl.program_id(0)

    @pl.when(i == 0)
    def _init():
        pmean = jnp.mean(pattern_ref[...], axis=0, keepdims=True)
        pmean_ref[...] = pmean
        z = jnp.mean(jnp.abs((pmean - mean_ref[...]) /
                             (jnp.sqrt(var_ref[...]) + 1e-6)))
        stat_sim = jnp.exp(-z * 0.5)
        h = (lax.dot_general(pmean, w1_ref[:, :DIM], (((1,), (1,)), ((), ())))
             + lax.dot_general(mean_ref[...], w1_ref[:, DIM:],
                               (((1,), (1,)), ((), ())))
             + b1_ref[...])
        h = jnp.maximum(h, 0.0)
        neural = jax.nn.sigmoid(jnp.sum(h * w2_ref[...]) + b2_ref[0])
        part_ref[0] = stat_sim * 0.4 + neural * 0.3
        min_ref[0] = jnp.inf

    diff = sp_ref[...] - pmean_ref[...]
    d2 = jnp.min(jnp.sum(diff * diff, axis=1))
    min_ref[0] = jnp.minimum(min_ref[0], d2)

    @pl.when(i == NB - 1)
    def _fini():
        nn_sim = jnp.exp(-jnp.sqrt(min_ref[0]))
        out_ref[0] = part_ref[0] + nn_sim * 0.3


def kernel(pattern, self_patterns, self_mean, self_var, W1, b1, W2, b2):
    out = pl.pallas_call(
        _body,
        grid=(NB,),
        in_specs=[
            pl.BlockSpec((1024, DIM), lambda i: (0, 0)),
            pl.BlockSpec((ROWS_PER_BLOCK, DIM), lambda i: (i, 0)),
            pl.BlockSpec((1, DIM), lambda i: (0, 0)),
            pl.BlockSpec((1, DIM), lambda i: (0, 0)),
            pl.BlockSpec((DIM, 2 * DIM), lambda i: (0, 0)),
            pl.BlockSpec((1, DIM), lambda i: (0, 0)),
            pl.BlockSpec((1, DIM), lambda i: (0, 0)),
            pl.BlockSpec(memory_space=pltpu.SMEM),
        ],
        out_specs=pl.BlockSpec(memory_space=pltpu.SMEM),
        out_shape=jax.ShapeDtypeStruct((1,), jnp.float32),
        scratch_shapes=[
            pltpu.VMEM((1, DIM), jnp.float32),
            pltpu.SMEM((1,), jnp.float32),
            pltpu.SMEM((1,), jnp.float32),
        ],
    )(pattern, self_patterns, self_mean.reshape(1, DIM),
      self_var.reshape(1, DIM), W1, b1.reshape(1, DIM), W2,
      b2)
    return out[0]
